# baseline (speedup 1.0000x reference)
import functools

import jax
import jax.numpy as jnp
from jax import lax
from jax.experimental import pallas as pl
from jax.experimental.pallas import tpu as pltpu
from jax.experimental.pallas import tpu_sc as plsc

R = 128
C = 32768
L = 16
NC = 2
NS = 16
NW = NC * NS
ROWS_PER_W = R // NW
CHUNKS = C // L
GRP = 16
GROUPS = CHUNKS // GRP
HALF = C // 2
ZC = 8192


def _permute(v, perm):
    dnums = lax.GatherDimensionNumbers(
        offset_dims=(), collapsed_slice_dims=(0,), start_index_map=(0,))
    return lax.gather(v, perm[:, None], dnums, (1,),
                      mode=lax.GatherScatterMode.PROMISE_IN_BOUNDS)


def _body(x_hbm, out_hbm, in0, in1, zero_v, hot4, sem_in, sem_z, sem_p):
    cid = lax.axis_index("c")
    sid = lax.axis_index("s")
    wid = sid * NC + cid
    row0 = wid * ROWS_PER_W

    lanes = lax.iota(jnp.int32, 16)
    zero = jnp.zeros((L,), jnp.float32)
    one = jnp.ones((L,), jnp.float32)
    neg_inf = jnp.full((L,), -jnp.inf, jnp.float32)
    izero = jnp.zeros((L,), jnp.int32)

    in_bufs = [in0, in1]
    pending = {(0, 0): pltpu.async_copy(
        x_hbm.at[row0, pl.ds(0, HALF)], in0.at[pl.ds(0, HALF)], sem_in),
        (0, 1): pltpu.async_copy(
        x_hbm.at[row0, pl.ds(HALF, HALF)], in0.at[pl.ds(HALF, HALF)], sem_in)}

    def zbody(i, carry):
        zero_v[pl.ds(i * L, L)] = zero
        return carry

    lax.fori_loop(0, ZC // L, zbody, 0, unroll=8)

    zcopies = [pltpu.async_copy(
        zero_v, out_hbm.at[row0 + r, pl.ds(k * ZC, ZC)], sem_z)
        for r in range(ROWS_PER_W) for k in range(C // ZC)]

    def scan_range(buf, g_lo, g_hi, carry):
        def g1(g, carry):
            m, gi = carry
            base = g * (GRP * L)
            accs = [buf[pl.ds(base + k * L, L)] for k in range(4)]
            for j in range(4, GRP):
                accs[j % 4] = jnp.maximum(accs[j % 4],
                                          buf[pl.ds(base + j * L, L)])
            gm = jnp.maximum(jnp.maximum(accs[0], accs[1]),
                             jnp.maximum(accs[2], accs[3]))
            upd = gm > m
            return jnp.where(upd, gm, m), jnp.where(upd, g, gi)

        return lax.fori_loop(g_lo, g_hi, g1, carry, unroll=4)

    pcopies = []
    for r in range(ROWS_PER_W):
        buf = in_bufs[r % 2]
        nbuf = in_bufs[(r + 1) % 2]

        pending.pop((r, 0)).wait()
        if r + 1 < ROWS_PER_W:
            pending[(r + 1, 0)] = pltpu.async_copy(
                x_hbm.at[row0 + r + 1, pl.ds(0, HALF)],
                nbuf.at[pl.ds(0, HALF)], sem_in)
        carry = scan_range(buf, 0, GROUPS // 2, (neg_inf, izero))

        pending.pop((r, 1)).wait()
        if r + 1 < ROWS_PER_W:
            pending[(r + 1, 1)] = pltpu.async_copy(
                x_hbm.at[row0 + r + 1, pl.ds(HALF, HALF)],
                nbuf.at[pl.ds(HALF, HALF)], sem_in)
        mvec, gvec = scan_range(buf, GROUPS // 2, GROUPS, carry)

        for s in (8, 4, 2, 1):
            perm = lanes ^ s
            om = _permute(mvec, perm)
            og = _permute(gvec, perm)
            take = (om > mvec) | ((om == mvec) & (og < gvec))
            mvec = jnp.where(take, om, mvec)
            gvec = jnp.where(take, og, gvec)

        gbase = gvec[0] * (GRP * L)
        big = jnp.full((L,), jnp.int32(2**30), jnp.int32)
        imin = big
        for j in range(GRP):
            v = buf[pl.ds(gbase + j * L, L)]
            idx = lanes + (gbase + j * L)
            imin = jnp.minimum(imin, jnp.where(v == mvec, idx, big))
        for s in (8, 4, 2, 1):
            imin = jnp.minimum(imin, _permute(imin, lanes ^ s))

        best = imin[0]
        blk = best // L
        lane = best - blk * L
        hot4[pl.ds(r * L, L)] = jnp.where(lanes == lane, one, zero)

        if r == 0:
            for zc in zcopies:
                zc.wait()
        pcopies.append(pltpu.async_copy(
            hot4.at[pl.ds(r * L, L)],
            out_hbm.at[row0 + r, pl.ds(blk * L, L)], sem_p))

    for pc in pcopies:
        pc.wait()


@jax.jit
def kernel(x):
    mesh = plsc.VectorSubcoreMesh(core_axis_name="c", subcore_axis_name="s")
    f = pl.kernel(
        _body,
        mesh=mesh,
        out_type=jax.ShapeDtypeStruct((R, C), jnp.float32),
        scratch_types=[
            pltpu.VMEM((C,), jnp.float32),
            pltpu.VMEM((C,), jnp.float32),
            pltpu.VMEM((ZC,), jnp.float32),
            pltpu.VMEM((ROWS_PER_W * L,), jnp.float32),
            pltpu.SemaphoreType.DMA,
            pltpu.SemaphoreType.DMA,
            pltpu.SemaphoreType.DMA,
        ],
    )
    return f(x)


# speedup vs baseline: 1.0485x; 1.0485x over previous
import functools

import jax
import jax.numpy as jnp
from jax import lax
from jax.experimental import pallas as pl
from jax.experimental.pallas import tpu as pltpu
from jax.experimental.pallas import tpu_sc as plsc

R = 128
C = 32768
L = 16
NC = 2
NS = 16
NW = NC * NS
ROWS_PER_W = R // NW
CHUNKS = C // L
GRP = 16
GROUPS = CHUNKS // GRP


def _permute(v, perm):
    dnums = lax.GatherDimensionNumbers(
        offset_dims=(), collapsed_slice_dims=(0,), start_index_map=(0,))
    return lax.gather(v, perm[:, None], dnums, (1,),
                      mode=lax.GatherScatterMode.PROMISE_IN_BOUNDS)


def _body(x_hbm, out_hbm, in0, in1, zero_v, hot4, sem_in, sem_z, sem_p):
    cid = lax.axis_index("c")
    sid = lax.axis_index("s")
    wid = sid * NC + cid
    row0 = wid * ROWS_PER_W

    lanes = lax.iota(jnp.int32, 16)
    zero = jnp.zeros((L,), jnp.float32)
    one = jnp.ones((L,), jnp.float32)
    neg_inf = jnp.full((L,), -jnp.inf, jnp.float32)
    izero = jnp.zeros((L,), jnp.int32)

    in_bufs = [in0, in1]
    first = pltpu.async_copy(x_hbm.at[row0], in0, sem_in)
    pending = {0: first}

    def zbody(i, carry):
        zero_v[pl.ds(i * L, L)] = zero
        return carry

    lax.fori_loop(0, CHUNKS, zbody, 0, unroll=8)

    zcopies = [pltpu.async_copy(zero_v, out_hbm.at[row0 + r], sem_z)
               for r in range(ROWS_PER_W)]

    blks = []
    for r in range(ROWS_PER_W):
        buf = in_bufs[r % 2]
        pending.pop(r).wait()
        if r + 1 < ROWS_PER_W:
            pending[r + 1] = pltpu.async_copy(
                x_hbm.at[row0 + r + 1], in_bufs[(r + 1) % 2], sem_in)

        def g1(g, carry):
            m, gi = carry
            base = g * (GRP * L)
            accs = [buf[pl.ds(base + k * L, L)] for k in range(4)]
            for j in range(4, GRP):
                accs[j % 4] = jnp.maximum(accs[j % 4],
                                          buf[pl.ds(base + j * L, L)])
            gm = jnp.maximum(jnp.maximum(accs[0], accs[1]),
                             jnp.maximum(accs[2], accs[3]))
            upd = gm > m
            return jnp.where(upd, gm, m), jnp.where(upd, g, gi)

        mvec, gvec = lax.fori_loop(
            0, GROUPS, g1, (neg_inf, izero), unroll=2)

        for s in (8, 4, 2, 1):
            perm = lanes ^ s
            om = _permute(mvec, perm)
            og = _permute(gvec, perm)
            take = (om > mvec) | ((om == mvec) & (og < gvec))
            mvec = jnp.where(take, om, mvec)
            gvec = jnp.where(take, og, gvec)

        gbase = gvec[0] * (GRP * L)
        big = jnp.full((L,), jnp.int32(2**30), jnp.int32)
        imin = big
        for j in range(GRP):
            v = buf[pl.ds(gbase + j * L, L)]
            idx = lanes + (gbase + j * L)
            imin = jnp.minimum(imin, jnp.where(v == mvec, idx, big))
        for s in (8, 4, 2, 1):
            imin = jnp.minimum(imin, _permute(imin, lanes ^ s))

        best = imin[0]
        blk = best // L
        lane = best - blk * L
        hot4[pl.ds(r * L, L)] = jnp.where(lanes == lane, one, zero)
        blks.append(blk)

    for zc in zcopies:
        zc.wait()
    pcopies = [pltpu.async_copy(
        hot4.at[pl.ds(r * L, L)],
        out_hbm.at[row0 + r, pl.ds(blks[r] * L, L)], sem_p)
        for r in range(ROWS_PER_W)]
    for pc in pcopies:
        pc.wait()


@jax.jit
def kernel(x):
    mesh = plsc.VectorSubcoreMesh(core_axis_name="c", subcore_axis_name="s")
    f = pl.kernel(
        _body,
        mesh=mesh,
        out_type=jax.ShapeDtypeStruct((R, C), jnp.float32),
        scratch_types=[
            pltpu.VMEM((C,), jnp.float32),
            pltpu.VMEM((C,), jnp.float32),
            pltpu.VMEM((C,), jnp.float32),
            pltpu.VMEM((ROWS_PER_W * L,), jnp.float32),
            pltpu.SemaphoreType.DMA,
            pltpu.SemaphoreType.DMA,
            pltpu.SemaphoreType.DMA,
        ],
    )
    return f(x)
